# baseline (speedup 1.0000x reference)
import functools

import jax
import jax.numpy as jnp
from jax import lax
from jax.experimental import pallas as pl
from jax.experimental.pallas import tpu as pltpu
from jax.experimental.pallas import tpu_sc as plsc

B = 16384
D = 32
R = 1000

_INFO = plsc.get_sparse_core_info()
_NC = _INFO.num_cores
_NS = _INFO.num_subcores
_NW = _NC * _NS
_BPW = B // _NW
_CHUNK = 128
_NCHUNK = _BPW // _CHUNK
_G = 16
_NG = _BPW // _G


def _body(entityT, relationT, subj2d, rel2d, obj2d, out_hbm,
          sidx_v, ridx_v, oidx_v, rbuf, e1buf, e2buf, out_v,
          sem, aux_sem):
    wid = lax.axis_index("c") * _NS + lax.axis_index("s")
    base = wid * _BPW

    row0 = wid * _NCHUNK
    pltpu.async_copy(subj2d.at[pl.ds(row0, _NCHUNK)], sidx_v, aux_sem).wait()
    pltpu.async_copy(obj2d.at[pl.ds(row0, _NCHUNK)], oidx_v, aux_sem).wait()
    pltpu.async_copy(rel2d.at[pl.ds(row0, _NCHUNK)], ridx_v, aux_sem).wait()

    descs = []
    for c in range(_NCHUNK):
        dst = pl.ds(c * _CHUNK, _CHUNK)
        for d in range(D):
            descs.append(pltpu.async_copy(entityT.at[d].at[sidx_v.at[c]],
                                          e1buf.at[d, dst], sem))
            descs.append(pltpu.async_copy(entityT.at[d].at[oidx_v.at[c]],
                                          e2buf.at[d, dst], sem))
            descs.append(pltpu.async_copy(relationT.at[d].at[ridx_v.at[c]],
                                          rbuf.at[d, dst], sem))
    for dsc in descs:
        dsc.wait()

    def compute_group(g, _):
        acc = jnp.zeros((_G,), jnp.float32)
        for d in range(D):
            acc += e1buf[d, pl.ds(g * _G, _G)] * rbuf[d, pl.ds(g * _G, _G)] \
                * e2buf[d, pl.ds(g * _G, _G)]
        out_v[pl.ds(g * _G, _G)] = acc
        return 0

    lax.fori_loop(0, _NG, compute_group, 0, unroll=False)

    pltpu.async_copy(out_v, out_hbm.at[pl.ds(base, _BPW)], aux_sem).wait()


@jax.jit
def _run(entityT, relationT, subj2d, rel2d, obj2d):
    mesh = plsc.VectorSubcoreMesh(core_axis_name="c", subcore_axis_name="s")
    kfn = pl.kernel(
        functools.partial(_body),
        out_type=jax.ShapeDtypeStruct((B,), jnp.float32),
        mesh=mesh,
        compiler_params=pltpu.CompilerParams(use_tc_tiling_on_sc=False),
        scratch_types=[
            pltpu.VMEM((_NCHUNK, _CHUNK), jnp.int32),
            pltpu.VMEM((_NCHUNK, _CHUNK), jnp.int32),
            pltpu.VMEM((_NCHUNK, _CHUNK), jnp.int32),
            pltpu.VMEM((D, _BPW), jnp.float32),
            pltpu.VMEM((D, _BPW), jnp.float32),
            pltpu.VMEM((D, _BPW), jnp.float32),
            pltpu.VMEM((_BPW,), jnp.float32),
            pltpu.SemaphoreType.DMA,
            pltpu.SemaphoreType.DMA,
        ],
    )
    return kfn(entityT, relationT, subj2d, rel2d, obj2d)


def kernel(entity_table, relation_table, subj_idx, rel_idx, obj_idx):
    entityT = entity_table.T
    relationT = relation_table.T
    subj2d = subj_idx.astype(jnp.int32).reshape(_NW * _NCHUNK, _CHUNK)
    rel2d = rel_idx.astype(jnp.int32).reshape(_NW * _NCHUNK, _CHUNK)
    obj2d = obj_idx.astype(jnp.int32).reshape(_NW * _NCHUNK, _CHUNK)
    return _run(entityT, relationT, subj2d, rel2d, obj2d)


# speedup vs baseline: 5.0987x; 5.0987x over previous
import functools

import jax
import jax.numpy as jnp
from jax import lax
from jax.experimental import pallas as pl
from jax.experimental.pallas import tpu as pltpu
from jax.experimental.pallas import tpu_sc as plsc

B = 16384
D = 32

_INFO = plsc.get_sparse_core_info()
_NC = _INFO.num_cores
_NS = _INFO.num_subcores
_NW = _NC * _NS
_BPW = B // _NW
_CHUNK = 128
_NCHUNK = _BPW // _CHUNK


def _body(subj2d, rel2d, obj2d, entity_hbm, relation_hbm, out_hbm,
          sidx_v, ridx_v, oidx_v, e1_v, r_v, e2_v, out_v, sem, idx_sem):
    wid = lax.axis_index("c") * _NS + lax.axis_index("s")
    base = wid * _BPW

    row0 = wid * _NCHUNK
    pltpu.async_copy(subj2d.at[pl.ds(row0, _NCHUNK)], sidx_v, idx_sem).wait()
    pltpu.async_copy(rel2d.at[pl.ds(row0, _NCHUNK)], ridx_v, idx_sem).wait()
    pltpu.async_copy(obj2d.at[pl.ds(row0, _NCHUNK)], oidx_v, idx_sem).wait()

    descs = []
    for j in range(_NCHUNK):
        dst = pl.ds(j * _CHUNK, _CHUNK)
        descs.append(pltpu.async_copy(entity_hbm.at[sidx_v.at[j]], e1_v.at[dst], sem))
        descs.append(pltpu.async_copy(relation_hbm.at[ridx_v.at[j]], r_v.at[dst], sem))
        descs.append(pltpu.async_copy(entity_hbm.at[oidx_v.at[j]], e2_v.at[dst], sem))
    for dsc in descs:
        dsc.wait()

    lane = lax.iota(jnp.int32, 16)
    bitrev = (((lane & 1) << 3) | ((lane & 2) << 1)
              | ((lane & 4) >> 1) | ((lane & 8) >> 3))

    _dnums = lax.GatherDimensionNumbers(
        offset_dims=(), collapsed_slice_dims=(0,), start_index_map=(0,))

    def shuf(v, idx):
        return lax.gather(v, idx[:, None], _dnums, (1,),
                          mode=lax.GatherScatterMode.PROMISE_IN_BOUNDS)

    def group(g, _):
        vecs = []
        for u in range(16):
            r = g * 16 + u
            vecs.append(
                e1_v[r, pl.ds(0, 16)] * r_v[r, pl.ds(0, 16)] * e2_v[r, pl.ds(0, 16)]
                + e1_v[r, pl.ds(16, 16)] * r_v[r, pl.ds(16, 16)] * e2_v[r, pl.ds(16, 16)])
        for k in (8, 4, 2, 1):
            m = (lane & k) == 0
            idx = lane ^ k
            vecs = [jnp.where(m, a + shuf(a, idx), b + shuf(b, idx))
                    for a, b in zip(vecs[0::2], vecs[1::2])]
        out_v[pl.ds(g * 16, 16)] = shuf(vecs[0], bitrev)
        return 0

    lax.fori_loop(0, _BPW // 16, group, 0, unroll=False)

    pltpu.async_copy(out_v, out_hbm.at[pl.ds(base, _BPW)], idx_sem).wait()


@jax.jit
def _run(entity_table, relation_table, subj2d, rel2d, obj2d):
    mesh = plsc.VectorSubcoreMesh(core_axis_name="c", subcore_axis_name="s")
    kfn = pl.kernel(
        functools.partial(_body),
        out_type=jax.ShapeDtypeStruct((B,), jnp.float32),
        mesh=mesh,
        compiler_params=pltpu.CompilerParams(use_tc_tiling_on_sc=False),
        scratch_types=[
            pltpu.VMEM((_NCHUNK, _CHUNK), jnp.int32),
            pltpu.VMEM((_NCHUNK, _CHUNK), jnp.int32),
            pltpu.VMEM((_NCHUNK, _CHUNK), jnp.int32),
            pltpu.VMEM((_BPW, D), jnp.float32),
            pltpu.VMEM((_BPW, D), jnp.float32),
            pltpu.VMEM((_BPW, D), jnp.float32),
            pltpu.VMEM((_BPW,), jnp.float32),
            pltpu.SemaphoreType.DMA,
            pltpu.SemaphoreType.DMA,
        ],
    )
    return kfn(subj2d, rel2d, obj2d, entity_table, relation_table)


def kernel(entity_table, relation_table, subj_idx, rel_idx, obj_idx):
    subj2d = subj_idx.astype(jnp.int32).reshape(_NW * _NCHUNK, _CHUNK)
    rel2d = rel_idx.astype(jnp.int32).reshape(_NW * _NCHUNK, _CHUNK)
    obj2d = obj_idx.astype(jnp.int32).reshape(_NW * _NCHUNK, _CHUNK)
    return _run(entity_table, relation_table, subj2d, rel2d, obj2d)


# speedup vs baseline: 6.8960x; 1.3525x over previous
import functools

import jax
import jax.numpy as jnp
from jax import lax
from jax.experimental import pallas as pl
from jax.experimental.pallas import tpu as pltpu
from jax.experimental.pallas import tpu_sc as plsc

B = 16384
D = 32

_INFO = plsc.get_sparse_core_info()
_NC = _INFO.num_cores
_NS = _INFO.num_subcores
_NW = _NC * _NS
_BPW = B // _NW
_CHUNK = 128
_NCHUNK = _BPW // _CHUNK
_G = 16
_NG = _BPW // _G


def _body(entity_hbm, relation_hbm, subj2d, rel2d, obj2d, out_hbm,
          sidx_v, ridx_v, oidx_v, e1b, rb, e2b, out_v, sem, aux_sem):
    wid = lax.axis_index("c") * _NS + lax.axis_index("s")
    base = wid * _BPW

    row0 = wid * _NCHUNK
    pltpu.async_copy(subj2d.at[pl.ds(row0, _NCHUNK)], sidx_v, aux_sem).wait()
    pltpu.async_copy(rel2d.at[pl.ds(row0, _NCHUNK)], ridx_v, aux_sem).wait()
    pltpu.async_copy(obj2d.at[pl.ds(row0, _NCHUNK)], oidx_v, aux_sem).wait()

    lane = lax.iota(jnp.int32, 16)
    bitrev = (((lane & 1) << 3) | ((lane & 2) << 1)
              | ((lane & 4) >> 1) | ((lane & 8) >> 3))
    _dnums = lax.GatherDimensionNumbers(
        offset_dims=(), collapsed_slice_dims=(0,), start_index_map=(0,))

    def shuf(v, idx):
        return lax.gather(v, idx[:, None], _dnums, (1,),
                          mode=lax.GatherScatterMode.PROMISE_IN_BOUNDS)

    def idx_vecs(g):
        gsl = (g >> 3, pl.ds((g & 7) * _G, _G))
        return sidx_v[gsl], ridx_v[gsl], oidx_v[gsl]

    def issue_group(g, slot):
        sv, rv, ov = idx_vecs(g)
        for u in range(_G):
            s8 = pl.multiple_of((sv[u] >> 3) * 8, 8)
            r8 = pl.multiple_of((rv[u] >> 3) * 8, 8)
            o8 = pl.multiple_of((ov[u] >> 3) * 8, 8)
            pltpu.async_copy(entity_hbm.at[pl.ds(s8, 8)], e1b.at[slot, u], sem)
            pltpu.async_copy(relation_hbm.at[pl.ds(r8, 8)], rb.at[slot, u], sem)
            pltpu.async_copy(entity_hbm.at[pl.ds(o8, 8)], e2b.at[slot, u], sem)

    def drain_group():
        for _ in range(_G):
            pltpu.make_async_copy(entity_hbm.at[pl.ds(0, 8)], e1b.at[0, 0], sem).wait()
            pltpu.make_async_copy(entity_hbm.at[pl.ds(0, 8)], rb.at[0, 0], sem).wait()
            pltpu.make_async_copy(entity_hbm.at[pl.ds(0, 8)], e2b.at[0, 0], sem).wait()

    def compute_group(g, slot):
        sv, rv, ov = idx_vecs(g)
        s7 = sv & 7
        r7 = rv & 7
        o7 = ov & 7
        vecs = []
        for u in range(_G):
            a = s7[u]
            b_ = r7[u]
            c_ = o7[u]
            vecs.append(
                e1b[slot, u, a, pl.ds(0, 16)] * rb[slot, u, b_, pl.ds(0, 16)]
                * e2b[slot, u, c_, pl.ds(0, 16)]
                + e1b[slot, u, a, pl.ds(16, 16)] * rb[slot, u, b_, pl.ds(16, 16)]
                * e2b[slot, u, c_, pl.ds(16, 16)])
        for k in (8, 4, 2, 1):
            m = (lane & k) == 0
            idx = lane ^ k
            vecs = [jnp.where(m, x + shuf(x, idx), y + shuf(y, idx))
                    for x, y in zip(vecs[0::2], vecs[1::2])]
        out_v[pl.ds(g * _G, _G)] = shuf(vecs[0], bitrev)

    issue_group(0, 0)

    def step(g, _):
        slot = g & 1
        issue_group(g + 1, 1 - slot)
        drain_group()
        compute_group(g, slot)
        return 0

    lax.fori_loop(0, _NG - 1, step, 0, unroll=False)
    drain_group()
    compute_group(_NG - 1, (_NG - 1) & 1)

    pltpu.async_copy(out_v, out_hbm.at[pl.ds(base, _BPW)], aux_sem).wait()


@jax.jit
def _run(entity_table, relation_table, subj2d, rel2d, obj2d):
    mesh = plsc.VectorSubcoreMesh(core_axis_name="c", subcore_axis_name="s")
    kfn = pl.kernel(
        functools.partial(_body),
        out_type=jax.ShapeDtypeStruct((B,), jnp.float32),
        mesh=mesh,
        scratch_types=[
            pltpu.VMEM((_NCHUNK, _CHUNK), jnp.int32),
            pltpu.VMEM((_NCHUNK, _CHUNK), jnp.int32),
            pltpu.VMEM((_NCHUNK, _CHUNK), jnp.int32),
            pltpu.VMEM((2, _G, 8, D), jnp.float32),
            pltpu.VMEM((2, _G, 8, D), jnp.float32),
            pltpu.VMEM((2, _G, 8, D), jnp.float32),
            pltpu.VMEM((_BPW,), jnp.float32),
            pltpu.SemaphoreType.DMA,
            pltpu.SemaphoreType.DMA,
        ],
    )
    return kfn(entity_table, relation_table, subj2d, rel2d, obj2d)


def kernel(entity_table, relation_table, subj_idx, rel_idx, obj_idx):
    subj2d = subj_idx.astype(jnp.int32).reshape(_NW * _NCHUNK, _CHUNK)
    rel2d = rel_idx.astype(jnp.int32).reshape(_NW * _NCHUNK, _CHUNK)
    obj2d = obj_idx.astype(jnp.int32).reshape(_NW * _NCHUNK, _CHUNK)
    return _run(entity_table, relation_table, subj2d, rel2d, obj2d)


# speedup vs baseline: 7.4442x; 1.0795x over previous
import functools

import jax
import jax.numpy as jnp
from jax import lax
from jax.experimental import pallas as pl
from jax.experimental.pallas import tpu as pltpu
from jax.experimental.pallas import tpu_sc as plsc

B = 16384
D = 32
R = 1000

_INFO = plsc.get_sparse_core_info()
_NC = _INFO.num_cores
_NS = _INFO.num_subcores
_NW = _NC * _NS
_BPW = B // _NW
_CHUNK = 128
_NCHUNK = _BPW // _CHUNK
_G = 16
_NG = _BPW // _G


def _body(entity_hbm, relation_hbm, subj2d, rel2d, obj2d, out_hbm,
          sidx_v, ridx_v, oidx_v, rel_v, e1b, e2b, out_v, sem, aux_sem):
    wid = lax.axis_index("c") * _NS + lax.axis_index("s")
    base = wid * _BPW

    reldsc = pltpu.async_copy(relation_hbm, rel_v, aux_sem)
    row0 = wid * _NCHUNK
    pltpu.async_copy(subj2d.at[pl.ds(row0, _NCHUNK)], sidx_v, aux_sem).wait()
    pltpu.async_copy(rel2d.at[pl.ds(row0, _NCHUNK)], ridx_v, aux_sem).wait()
    pltpu.async_copy(obj2d.at[pl.ds(row0, _NCHUNK)], oidx_v, aux_sem).wait()
    reldsc.wait()

    lane = lax.iota(jnp.int32, 16)
    bitrev = (((lane & 1) << 3) | ((lane & 2) << 1)
              | ((lane & 4) >> 1) | ((lane & 8) >> 3))
    _dnums = lax.GatherDimensionNumbers(
        offset_dims=(), collapsed_slice_dims=(0,), start_index_map=(0,))

    def shuf(v, idx):
        return lax.gather(v, idx[:, None], _dnums, (1,),
                          mode=lax.GatherScatterMode.PROMISE_IN_BOUNDS)

    def idx_vecs(g):
        gsl = (g >> 3, pl.ds((g & 7) * _G, _G))
        return sidx_v[gsl], ridx_v[gsl], oidx_v[gsl]

    def issue_group(g, slot):
        sv, _, ov = idx_vecs(g)
        for u in range(_G):
            s8 = pl.multiple_of((sv[u] >> 3) * 8, 8)
            o8 = pl.multiple_of((ov[u] >> 3) * 8, 8)
            pltpu.async_copy(entity_hbm.at[pl.ds(s8, 8)],
                             e1b.at[slot, u], sem)
            pltpu.async_copy(entity_hbm.at[pl.ds(o8, 8)],
                             e2b.at[slot, u], sem)

    def drain_group():
        for _ in range(_G):
            pltpu.make_async_copy(entity_hbm.at[pl.ds(0, 8)], e1b.at[0, 0], sem).wait()
            pltpu.make_async_copy(entity_hbm.at[pl.ds(0, 8)], e2b.at[0, 0], sem).wait()

    def compute_group(g, slot):
        sv, rv, ov = idx_vecs(g)
        s7 = sv & 7
        o7 = ov & 7
        vecs = []
        for u in range(_G):
            a = s7[u]
            rq = rv[u] >> 2
            ro = (rv[u] & 3) * 32
            c_ = o7[u]
            vecs.append(
                e1b[slot, u, a, pl.ds(0, 16)] * rel_v[rq, pl.ds(ro, 16)]
                * e2b[slot, u, c_, pl.ds(0, 16)]
                + e1b[slot, u, a, pl.ds(16, 16)] * rel_v[rq, pl.ds(ro + 16, 16)]
                * e2b[slot, u, c_, pl.ds(16, 16)])
        for k in (8, 4, 2, 1):
            m = (lane & k) == 0
            idx = lane ^ k
            vecs = [jnp.where(m, x + shuf(x, idx), y + shuf(y, idx))
                    for x, y in zip(vecs[0::2], vecs[1::2])]
        out_v[pl.ds(g * _G, _G)] = shuf(vecs[0], bitrev)

    issue_group(0, 0)

    def step(g, _):
        slot = g & 1
        issue_group(g + 1, 1 - slot)
        drain_group()
        compute_group(g, slot)
        return 0

    lax.fori_loop(0, _NG - 1, step, 0, unroll=False)
    drain_group()
    compute_group(_NG - 1, (_NG - 1) & 1)

    pltpu.async_copy(out_v, out_hbm.at[pl.ds(base, _BPW)], aux_sem).wait()


@jax.jit
def _run(entity_table, relation_table, subj2d, rel2d, obj2d):
    mesh = plsc.VectorSubcoreMesh(core_axis_name="c", subcore_axis_name="s")
    kfn = pl.kernel(
        functools.partial(_body),
        out_type=jax.ShapeDtypeStruct((B,), jnp.float32),
        mesh=mesh,
        scratch_types=[
            pltpu.VMEM((_NCHUNK, _CHUNK), jnp.int32),
            pltpu.VMEM((_NCHUNK, _CHUNK), jnp.int32),
            pltpu.VMEM((_NCHUNK, _CHUNK), jnp.int32),
            pltpu.VMEM((R // 4, 128), jnp.float32),
            pltpu.VMEM((2, _G, 8, D), jnp.float32),
            pltpu.VMEM((2, _G, 8, D), jnp.float32),
            pltpu.VMEM((_BPW,), jnp.float32),
            pltpu.SemaphoreType.DMA,
            pltpu.SemaphoreType.DMA,
        ],
    )
    return kfn(entity_table, relation_table, subj2d, rel2d, obj2d)


def kernel(entity_table, relation_table, subj_idx, rel_idx, obj_idx):
    relation128 = relation_table.reshape(R // 4, 128)
    subj2d = subj_idx.astype(jnp.int32).reshape(_NW * _NCHUNK, _CHUNK)
    rel2d = rel_idx.astype(jnp.int32).reshape(_NW * _NCHUNK, _CHUNK)
    obj2d = obj_idx.astype(jnp.int32).reshape(_NW * _NCHUNK, _CHUNK)
    return _run(entity_table, relation128, subj2d, rel2d, obj2d)


# speedup vs baseline: 11.6422x; 1.5639x over previous
import functools

import jax
import jax.numpy as jnp
from jax import lax
from jax.experimental import pallas as pl
from jax.experimental.pallas import tpu as pltpu
from jax.experimental.pallas import tpu_sc as plsc

B = 16384
D = 32
R = 1000

_INFO = plsc.get_sparse_core_info()
_NC = _INFO.num_cores
_NS = _INFO.num_subcores
_NW = _NC * _NS
_BPW = B // _NW
_CHUNK = 128
_NCHUNK = _BPW // _CHUNK
_G = 16
_NG = _BPW // _G


def _body(entity_hbm, relation_hbm, subj2d, rel2d, obj2d, out_hbm,
          sidx_v, ridx_v, oidx_v, rel_v, e1b, e2b, out_v, sem, aux_sem):
    wid = lax.axis_index("c") * _NS + lax.axis_index("s")
    base = wid * _BPW

    reldsc = pltpu.async_copy(relation_hbm, rel_v, aux_sem)
    row0 = wid * _NCHUNK
    pltpu.async_copy(subj2d.at[pl.ds(row0, _NCHUNK)], sidx_v, aux_sem).wait()
    pltpu.async_copy(rel2d.at[pl.ds(row0, _NCHUNK)], ridx_v, aux_sem).wait()
    pltpu.async_copy(obj2d.at[pl.ds(row0, _NCHUNK)], oidx_v, aux_sem).wait()
    reldsc.wait()

    lane = lax.iota(jnp.int32, 16)
    bitrev = (((lane & 1) << 3) | ((lane & 2) << 1)
              | ((lane & 4) >> 1) | ((lane & 8) >> 3))
    _dnums = lax.GatherDimensionNumbers(
        offset_dims=(), collapsed_slice_dims=(0,), start_index_map=(0,))

    def shuf(v, idx):
        return lax.gather(v, idx[:, None], _dnums, (1,),
                          mode=lax.GatherScatterMode.PROMISE_IN_BOUNDS)

    def idx_vecs(g):
        gsl = (g >> 3, pl.ds((g & 7) * _G, _G))
        return sidx_v[gsl], ridx_v[gsl], oidx_v[gsl]

    def issue_group(g, slot):
        sv, _, ov = idx_vecs(g)
        for u in range(_G):
            pltpu.async_copy(entity_hbm.at[sv[u] >> 3], e1b.at[slot, u], sem)
            pltpu.async_copy(entity_hbm.at[ov[u] >> 3], e2b.at[slot, u], sem)

    def drain_group():
        for _ in range(_G):
            pltpu.make_async_copy(entity_hbm.at[0], e1b.at[0, 0], sem).wait()
            pltpu.make_async_copy(entity_hbm.at[0], e2b.at[0, 0], sem).wait()

    def compute_group(g, slot):
        sv, rv, ov = idx_vecs(g)
        s7 = sv & 7
        o7 = ov & 7
        vecs = []
        for u in range(_G):
            a = s7[u]
            rq = rv[u] >> 2
            ro = (rv[u] & 3) * 32
            c_ = o7[u]
            vecs.append(
                e1b[slot, u, a, pl.ds(0, 16)] * rel_v[rq, pl.ds(ro, 16)]
                * e2b[slot, u, c_, pl.ds(0, 16)]
                + e1b[slot, u, a, pl.ds(16, 16)] * rel_v[rq, pl.ds(ro + 16, 16)]
                * e2b[slot, u, c_, pl.ds(16, 16)])
        for k in (8, 4, 2, 1):
            m = (lane & k) == 0
            idx = lane ^ k
            vecs = [jnp.where(m, x + shuf(x, idx), y + shuf(y, idx))
                    for x, y in zip(vecs[0::2], vecs[1::2])]
        out_v[pl.ds(g * _G, _G)] = shuf(vecs[0], bitrev)

    issue_group(0, 0)

    def step(g, _):
        slot = g & 1
        issue_group(g + 1, 1 - slot)
        drain_group()
        compute_group(g, slot)
        return 0

    lax.fori_loop(0, _NG - 1, step, 0, unroll=False)
    drain_group()
    compute_group(_NG - 1, (_NG - 1) & 1)

    pltpu.async_copy(out_v, out_hbm.at[pl.ds(base, _BPW)], aux_sem).wait()


@jax.jit
def _run(entity_table, relation_table, subj2d, rel2d, obj2d):
    mesh = plsc.VectorSubcoreMesh(core_axis_name="c", subcore_axis_name="s")
    kfn = pl.kernel(
        functools.partial(_body),
        out_type=jax.ShapeDtypeStruct((B,), jnp.float32),
        mesh=mesh,
        scratch_types=[
            pltpu.VMEM((_NCHUNK, _CHUNK), jnp.int32),
            pltpu.VMEM((_NCHUNK, _CHUNK), jnp.int32),
            pltpu.VMEM((_NCHUNK, _CHUNK), jnp.int32),
            pltpu.VMEM((R // 4, 128), jnp.float32),
            pltpu.VMEM((2, _G, 8, D), jnp.float32),
            pltpu.VMEM((2, _G, 8, D), jnp.float32),
            pltpu.VMEM((_BPW,), jnp.float32),
            pltpu.SemaphoreType.DMA,
            pltpu.SemaphoreType.DMA,
        ],
    )
    return kfn(entity_table, relation_table, subj2d, rel2d, obj2d)


def kernel(entity_table, relation_table, subj_idx, rel_idx, obj_idx):
    entity3 = entity_table.reshape(-1, 8, D)
    relation128 = relation_table.reshape(R // 4, 128)
    subj2d = subj_idx.astype(jnp.int32).reshape(_NW * _NCHUNK, _CHUNK)
    rel2d = rel_idx.astype(jnp.int32).reshape(_NW * _NCHUNK, _CHUNK)
    obj2d = obj_idx.astype(jnp.int32).reshape(_NW * _NCHUNK, _CHUNK)
    return _run(entity3, relation128, subj2d, rel2d, obj2d)
